# baseline (speedup 1.0000x reference)
import jax
import jax.numpy as jnp
from jax.experimental import pallas as pl
from jax.experimental.pallas import tpu as pltpu

N = 4096
D = 256
K = 64
BQ = 512


def _body(x_ref, w_ref, a_ref, nr_ref, nc_ref, y_ref, o_s):
    i = pl.program_id(0)

    @pl.when(i == 0)
    def _():
        fw = jax.nn.softmax(a_ref[...], axis=-1)
        proj = jax.lax.dot_general(
            x_ref[...], w_ref[...], (((1,), (1,)), ((), ())),
            preferred_element_type=jnp.float32)
        o_s[...] = proj * fw

    xb = x_ref[pl.ds(i * BQ, BQ), :]
    raw = jax.lax.dot_general(
        xb, x_ref[...], (((1,), (1,)), ((), ())),
        preferred_element_type=jnp.float32)
    s = raw / (nc_ref[...] * nr_ref[...])

    m = jnp.float32(1.0)

    bits = jax.lax.bitcast_convert_type(s, jnp.int32)
    key = jnp.where(bits >= 0, bits, jnp.int32(-2147483648) - bits)

    h16 = (key >> 16).astype(jnp.int16)
    l16 = key.astype(jnp.int16) ^ jnp.int16(-32768)

    ones8 = jnp.ones((128, 8), jnp.bfloat16)

    def count_ge16(data16, mid):
        mid16 = mid.astype(jnp.int16)
        one = jnp.bfloat16(1)
        zero = jnp.bfloat16(0)
        acc = jnp.where(data16[:, :128] >= mid16, one, zero)
        for g in range(1, N // 128):
            acc = acc + jnp.where(data16[:, g * 128:(g + 1) * 128] >= mid16, one, zero)
        cnt = jax.lax.dot_general(acc, ones8, (((1,), (0,)), ((), ())),
                                  preferred_element_type=jnp.float32)
        return cnt[:, :1]

    lo = jnp.full((BQ, 1), -16384, jnp.int32)
    hi = jnp.full((BQ, 1), 16384, jnp.int32)
    cnt_hi = jnp.zeros((BQ, 1), jnp.float32)
    for _ in range(15):
        mid = lo + ((hi - lo) >> 1)
        cnt = count_ge16(h16, mid)
        pred = cnt >= K
        lo = jnp.where(pred, mid, lo)
        hi = jnp.where(pred, hi, mid)
        cnt_hi = jnp.where(pred, cnt_hi, cnt)
    hstar = lo
    hstar16 = hstar.astype(jnp.int16)

    kp = K - cnt_hi
    active = h16 == hstar16
    lmask = jnp.where(active, l16, jnp.int16(-32768))

    lo = jnp.full((BQ, 1), -32768, jnp.int32)
    hi = jnp.full((BQ, 1), 32768, jnp.int32)
    for _ in range(17):
        mid = lo + ((hi - lo) >> 1)
        pred = count_ge16(lmask, mid) >= kp
        lo = jnp.where(pred, mid, lo)
        hi = jnp.where(pred, hi, mid)
    lstar = lo

    key_thr = (hstar << 16) | ((lstar ^ 32768) & 0xFFFF)
    thr_f = jax.lax.bitcast_convert_type(
        jnp.where(key_thr >= 0, key_thr, jnp.int32(-2147483648) - key_thr),
        jnp.float32)

    p = jnp.where(s >= thr_f, jnp.exp(s - m), 0.0)
    z = jnp.sum(p, axis=-1, keepdims=True)
    y_ref[...] = jax.lax.dot_general(
        p, o_s[...], (((1,), (0,)), ((), ())),
        preferred_element_type=jnp.float32) / z


def kernel(x, W, a):
    a2 = a.reshape(1, D)
    nrm = jnp.linalg.norm(x, axis=-1)
    nrow = nrm.reshape(1, N)
    ncol = nrm.reshape(N, 1)
    return pl.pallas_call(
        _body,
        grid=(N // BQ,),
        in_specs=[
            pl.BlockSpec((N, D), lambda i: (0, 0)),
            pl.BlockSpec((D, D), lambda i: (0, 0)),
            pl.BlockSpec((1, D), lambda i: (0, 0)),
            pl.BlockSpec((1, N), lambda i: (0, 0)),
            pl.BlockSpec((BQ, 1), lambda i: (i, 0)),
        ],
        out_specs=pl.BlockSpec((BQ, D), lambda i: (i, 0)),
        out_shape=jax.ShapeDtypeStruct((N, D), jnp.float32),
        scratch_shapes=[
            pltpu.VMEM((N, D), jnp.float32),
        ],
    )(x, W, a2, nrow, ncol)


# speedup vs baseline: 1.0007x; 1.0007x over previous
import jax
import jax.numpy as jnp
from jax.experimental import pallas as pl
from jax.experimental.pallas import tpu as pltpu

N = 4096
D = 256
K = 64
BQ = 512


def _body(x_ref, w_ref, a_ref, nr_ref, nc_ref, y_ref, o_s):
    i = pl.program_id(0)

    @pl.when(i == 0)
    def _():
        fw = jax.nn.softmax(a_ref[...], axis=-1)
        proj = jax.lax.dot_general(
            x_ref[...], w_ref[...], (((1,), (1,)), ((), ())),
            preferred_element_type=jnp.float32)
        o_s[...] = proj * fw

    xb = x_ref[pl.ds(i * BQ, BQ), :]
    raw = jax.lax.dot_general(
        xb, x_ref[...], (((1,), (1,)), ((), ())),
        preferred_element_type=jnp.float32)
    s = raw / (nc_ref[...] * nr_ref[...])

    m = jnp.float32(1.0)

    bits = jax.lax.bitcast_convert_type(s, jnp.int32)
    key = jnp.where(bits >= 0, bits, jnp.int32(-2147483648) - bits)

    h16 = (key >> 16).astype(jnp.int16)

    ones8 = jnp.ones((128, 8), jnp.bfloat16)

    def count_ge16(data16, mid):
        mid16 = mid.astype(jnp.int16)
        one = jnp.bfloat16(1)
        zero = jnp.bfloat16(0)
        acc = jnp.where(data16[:, :128] >= mid16, one, zero)
        for g in range(1, N // 128):
            acc = acc + jnp.where(data16[:, g * 128:(g + 1) * 128] >= mid16, one, zero)
        cnt = jax.lax.dot_general(acc, ones8, (((1,), (0,)), ((), ())),
                                  preferred_element_type=jnp.float32)
        return cnt[:, :1]

    lo = jnp.full((BQ, 1), -16384, jnp.int32)
    hi = jnp.full((BQ, 1), 16384, jnp.int32)
    cnt_hi = jnp.zeros((BQ, 1), jnp.float32)
    for _ in range(15):
        mid = lo + ((hi - lo) >> 1)
        cnt = count_ge16(h16, mid)
        pred = cnt >= K
        lo = jnp.where(pred, mid, lo)
        hi = jnp.where(pred, hi, mid)
        cnt_hi = jnp.where(pred, cnt_hi, cnt)
    hstar = lo
    hstar16 = hstar.astype(jnp.int16)

    kp = K - cnt_hi
    active = h16 == hstar16
    l16 = key.astype(jnp.int16) ^ jnp.int16(-32768)
    lmask = jnp.where(active, l16, jnp.int16(-32768))

    lo = jnp.full((BQ, 1), -32768, jnp.int32)
    hi = jnp.full((BQ, 1), 32768, jnp.int32)
    for _ in range(17):
        mid = lo + ((hi - lo) >> 1)
        pred = count_ge16(lmask, mid) >= kp
        lo = jnp.where(pred, mid, lo)
        hi = jnp.where(pred, hi, mid)
    lstar = lo

    key_thr = (hstar << 16) | ((lstar ^ 32768) & 0xFFFF)
    thr_f = jax.lax.bitcast_convert_type(
        jnp.where(key_thr >= 0, key_thr, jnp.int32(-2147483648) - key_thr),
        jnp.float32)

    p = jnp.where(s >= thr_f, jnp.exp(s - m), 0.0)
    z = jnp.sum(p, axis=-1, keepdims=True)
    y_ref[...] = jax.lax.dot_general(
        p, o_s[...], (((1,), (0,)), ((), ())),
        preferred_element_type=jnp.float32) / z


def kernel(x, W, a):
    a2 = a.reshape(1, D)
    nrm = jnp.linalg.norm(x, axis=-1)
    nrow = nrm.reshape(1, N)
    ncol = nrm.reshape(N, 1)
    return pl.pallas_call(
        _body,
        grid=(N // BQ,),
        in_specs=[
            pl.BlockSpec((N, D), lambda i: (0, 0)),
            pl.BlockSpec((D, D), lambda i: (0, 0)),
            pl.BlockSpec((1, D), lambda i: (0, 0)),
            pl.BlockSpec((1, N), lambda i: (0, 0)),
            pl.BlockSpec((BQ, 1), lambda i: (i, 0)),
        ],
        out_specs=pl.BlockSpec((BQ, D), lambda i: (i, 0)),
        out_shape=jax.ShapeDtypeStruct((N, D), jnp.float32),
        scratch_shapes=[
            pltpu.VMEM((N, D), jnp.float32),
        ],
    )(x, W, a2, nrow, ncol)


# speedup vs baseline: 1.0022x; 1.0016x over previous
import jax
import jax.numpy as jnp
from jax.experimental import pallas as pl
from jax.experimental.pallas import tpu as pltpu

N = 4096
D = 256
K = 64
BQ = 512


def _body(x_ref, w_ref, a_ref, nr_ref, nc_ref, y_ref, o_s):
    i = pl.program_id(0)

    @pl.when(i == 0)
    def _():
        fw = jax.nn.softmax(a_ref[...], axis=-1)
        proj = jax.lax.dot_general(
            x_ref[...], w_ref[...], (((1,), (1,)), ((), ())),
            preferred_element_type=jnp.float32)
        o_s[...] = proj * fw

    xb = x_ref[pl.ds(i * BQ, BQ), :]
    raw = jax.lax.dot_general(
        xb, x_ref[...], (((1,), (1,)), ((), ())),
        preferred_element_type=jnp.float32)
    s = raw / (nc_ref[...] * nr_ref[...])

    m = jnp.float32(1.0)

    bits = jax.lax.bitcast_convert_type(s, jnp.int32)
    key = jnp.where(bits >= 0, bits, jnp.int32(-2147483648) - bits)

    h16 = (key >> 16).astype(jnp.int16)

    ones8 = jnp.ones((128, 8), jnp.bfloat16)

    def count_ge16(data16, mid):
        mid16 = mid.astype(jnp.int16)
        one = jnp.bfloat16(1)
        zero = jnp.bfloat16(0)
        acc = jnp.where(data16[:, :128] >= mid16, one, zero)
        for g in range(1, N // 128):
            acc = acc + jnp.where(data16[:, g * 128:(g + 1) * 128] >= mid16, one, zero)
        cnt = jax.lax.dot_general(acc, ones8, (((1,), (0,)), ((), ())),
                                  preferred_element_type=jnp.float32)
        return cnt[:, :1]

    lo = jnp.full((BQ, 1), -16384, jnp.int32)
    hi = jnp.full((BQ, 1), 16384, jnp.int32)
    cnt_hi = jnp.zeros((BQ, 1), jnp.float32)
    for _ in range(15):
        mid = lo + ((hi - lo) >> 1)
        cnt = count_ge16(h16, mid)
        pred = cnt >= K
        lo = jnp.where(pred, mid, lo)
        hi = jnp.where(pred, hi, mid)
        cnt_hi = jnp.where(pred, cnt_hi, cnt)
    hstar = lo
    hstar16 = hstar.astype(jnp.int16)

    kp = K - cnt_hi
    active = h16 == hstar16
    l16 = key.astype(jnp.int16) ^ jnp.int16(-32768)
    lmask = jnp.where(active, l16, jnp.int16(-32768))

    lo = jnp.full((BQ, 1), -32768, jnp.int32)
    hi = jnp.full((BQ, 1), 32768, jnp.int32)
    for _ in range(16):
        mid = lo + ((hi - lo) >> 1)
        pred = count_ge16(lmask, mid) >= kp
        lo = jnp.where(pred, mid, lo)
        hi = jnp.where(pred, hi, mid)
    lstar = lo

    key_thr = (hstar << 16) | ((lstar ^ 32768) & 0xFFFF)
    thr_f = jax.lax.bitcast_convert_type(
        jnp.where(key_thr >= 0, key_thr, jnp.int32(-2147483648) - key_thr),
        jnp.float32)

    p = jnp.where(s >= thr_f, jnp.exp(s - m), 0.0)
    z = jnp.sum(p, axis=-1, keepdims=True)
    y_ref[...] = jax.lax.dot_general(
        p, o_s[...], (((1,), (0,)), ((), ())),
        preferred_element_type=jnp.float32) / z


def kernel(x, W, a):
    a2 = a.reshape(1, D)
    nrm = jnp.linalg.norm(x, axis=-1)
    nrow = nrm.reshape(1, N)
    ncol = nrm.reshape(N, 1)
    return pl.pallas_call(
        _body,
        grid=(N // BQ,),
        in_specs=[
            pl.BlockSpec((N, D), lambda i: (0, 0)),
            pl.BlockSpec((D, D), lambda i: (0, 0)),
            pl.BlockSpec((1, D), lambda i: (0, 0)),
            pl.BlockSpec((1, N), lambda i: (0, 0)),
            pl.BlockSpec((BQ, 1), lambda i: (i, 0)),
        ],
        out_specs=pl.BlockSpec((BQ, D), lambda i: (i, 0)),
        out_shape=jax.ShapeDtypeStruct((N, D), jnp.float32),
        scratch_shapes=[
            pltpu.VMEM((N, D), jnp.float32),
        ],
    )(x, W, a2, nrow, ncol)
